# baseline (speedup 1.0000x reference)
import functools

import jax
import jax.numpy as jnp
from jax import lax
from jax.experimental import pallas as pl
from jax.experimental.pallas import tpu as pltpu
from jax.experimental.pallas import tpu_sc as plsc

E = 8
TOP_K = 2
D = 768
FF = 3072
T = 2048
CAP = int(T * TOP_K / E * 1.25)
SLOTS = E * CAP
SLOTS_PAD = SLOTS + 16

DP = D // 2
DB = 256
NJ2 = D // DB

NC, NS = 2, 16
NW = NC * NS
TPW = T // NW
TCHUNK = 16


def _routing_kernel(x_ref, wg_ref, lng_ref, lnb_ref,
                    h_ref, d1_ref, d2_ref, w1_ref, w2_ref):
    x = x_ref[...]
    mu = jnp.mean(x, axis=-1, keepdims=True)
    var = jnp.mean((x - mu) ** 2, axis=-1, keepdims=True)
    h = (x - mu) / jnp.sqrt(var + 1e-5) * lng_ref[...] + lnb_ref[...]
    h_ref[...] = h

    logits = jnp.dot(h.astype(jnp.bfloat16), wg_ref[...].astype(jnp.bfloat16),
                     preferred_element_type=jnp.float32)
    m = jnp.max(logits, axis=-1, keepdims=True)
    ex = jnp.exp(logits - m)
    probs = ex / jnp.sum(ex, axis=-1, keepdims=True)

    iota_e = lax.broadcasted_iota(jnp.int32, (T, E), 1)
    m1 = jnp.max(probs, axis=-1, keepdims=True)
    i1 = jnp.min(jnp.where(probs == m1, iota_e, E), axis=-1, keepdims=True)
    probs2 = jnp.where(iota_e == i1, -1.0, probs)
    m2 = jnp.max(probs2, axis=-1, keepdims=True)
    i2 = jnp.min(jnp.where(probs2 == m2, iota_e, E), axis=-1, keepdims=True)

    s = jnp.clip(m1 + m2, 1e-9, None)
    g1 = m1 / s
    g2 = m2 / s

    oh1 = (iota_e == i1).astype(jnp.float32)
    oh2 = (iota_e == i2).astype(jnp.float32)
    cnt = oh1 + oh2

    r = lax.broadcasted_iota(jnp.int32, (128, 128), 0)
    c = lax.broadcasted_iota(jnp.int32, (128, 128), 1)
    ltri = (c < r).astype(jnp.float32)
    segs = []
    run = jnp.zeros((1, E), dtype=jnp.float32)
    for b in range(T // 128):
        blk = cnt[b * 128:(b + 1) * 128, :]
        excl = jnp.dot(ltri, blk, preferred_element_type=jnp.float32)
        segs.append(excl + run)
        run = run + jnp.sum(blk, axis=0, keepdims=True)
    pos = jnp.concatenate(segs, axis=0)

    p1 = jnp.sum(oh1 * pos, axis=-1, keepdims=True)
    p2 = jnp.sum(oh2 * pos, axis=-1, keepdims=True)
    keep1 = p1 < CAP
    keep2 = p2 < CAP

    slot1 = i1 * CAP + p1.astype(jnp.int32)
    slot2 = i2 * CAP + p2.astype(jnp.int32)
    d1_ref[...] = jnp.where(keep1, slot1, -1)
    d2_ref[...] = jnp.where(keep2, slot2, -1)
    w1_ref[...] = jnp.where(keep1, g1, 0.0)
    w2_ref[...] = jnp.where(keep2, g2, 0.0)


def _dispatch_body(h_hbm, d1_hbm, d2_hbm, xin_hbm, d1_v, d2_v, rows_v, sem):
    wid = lax.axis_index("s") * NC + lax.axis_index("c")
    tb = wid * TPW
    c1 = pltpu.async_copy(d1_hbm.at[pl.ds(tb, TPW)], d1_v, sem)
    c2 = pltpu.async_copy(d2_hbm.at[pl.ds(tb, TPW)], d2_v, sem)
    c3 = pltpu.async_copy(h_hbm.at[pl.ds(tb, TPW)], rows_v, sem)
    c1.wait()
    c2.wait()
    c3.wait()
    for i in range(TPW // 16):
        sl = pl.ds(i * 16, 16)
        v1 = d1_v[sl]
        d1_v[sl] = jnp.where(v1 < 0, SLOTS, v1)
        v2 = d2_v[sl]
        d2_v[sl] = jnp.where(v2 < 0, SLOTS, v2)
    s1 = pltpu.async_copy(rows_v, xin_hbm.at[d1_v], sem)
    s2 = pltpu.async_copy(rows_v, xin_hbm.at[d2_v], sem)
    s1.wait()
    s2.wait()


def _ffn_kernel(xin_ref, w1_ref, b1_ref, w2_ref, b2_ref, y_ref, hid_ref):
    hid = jnp.dot(xin_ref[...].astype(jnp.bfloat16),
                  w1_ref[0].astype(jnp.bfloat16),
                  preferred_element_type=jnp.float32) + b1_ref[0]
    hid_ref[...] = jnp.maximum(hid, 0.0).astype(jnp.bfloat16)
    y_ref[...] = jnp.dot(hid_ref[...], w2_ref[0].astype(jnp.bfloat16),
                         preferred_element_type=jnp.float32) + b2_ref[0]


def _combine_body(y_hbm, x_hbm, dg1_hbm, dg2_hbm, w1_hbm, w2_hbm, out_hbm,
                  dg1_v, dg2_v, w1_v, w2_v, r1_v, r2_v, xb_v,
                  gsem0, gsem1, wsem):
    wid = lax.axis_index("s") * NC + lax.axis_index("c")
    tb = wid * TPW
    pltpu.sync_copy(dg1_hbm.at[pl.ds(tb, TPW)], dg1_v)
    pltpu.sync_copy(dg2_hbm.at[pl.ds(tb, TPW)], dg2_v)
    pltpu.sync_copy(w1_hbm.at[pl.ds(tb, TPW)], w1_v)
    pltpu.sync_copy(w2_hbm.at[pl.ds(tb, TPW)], w2_v)
    for i in range(TPW // 16):
        sl = pl.ds(i * 16, 16)
        v1 = dg1_v[sl]
        dg1_v[sl] = jnp.where(v1 < 0, 0, v1)
        v2 = dg2_v[sl]
        dg2_v[sl] = jnp.where(v2 < 0, 0, v2)

    def fire(c):
        slot = c % 2
        sem = gsem0 if slot == 0 else gsem1
        return [
            pltpu.async_copy(y_hbm.at[dg1_v.at[pl.ds(c * TCHUNK, TCHUNK)]],
                             r1_v.at[slot], sem),
            pltpu.async_copy(y_hbm.at[dg2_v.at[pl.ds(c * TCHUNK, TCHUNK)]],
                             r2_v.at[slot], sem),
            pltpu.async_copy(x_hbm.at[pl.ds(tb + c * TCHUNK, TCHUNK)],
                             xb_v.at[slot], sem),
        ]

    nch = TPW // TCHUNK
    pend = fire(0)
    wpend = None
    for c in range(nch):
        slot = c % 2
        for d in pend:
            d.wait()
        if wpend is not None:
            wpend.wait()
            wpend = None
        if c + 1 < nch:
            pend = fire(c + 1)

        def tok_body(tloc, carry):
            sidx = jnp.zeros((16,), jnp.int32) + (c * TCHUNK + tloc)
            w1s = plsc.load_gather(w1_v, [sidx])
            w2s = plsc.load_gather(w2_v, [sidx])
            for v in range(D // 16):
                sl = pl.ds(v * 16, 16)
                acc = (xb_v[slot, tloc, sl] + w1s * r1_v[slot, tloc, sl]
                       + w2s * r2_v[slot, tloc, sl])
                xb_v[slot, tloc, sl] = acc
            return carry

        lax.fori_loop(0, TCHUNK, tok_body, 0)
        wpend = pltpu.async_copy(xb_v.at[slot],
                                 out_hbm.at[pl.ds(tb + c * TCHUNK, TCHUNK)],
                                 wsem)
    wpend.wait()


_SC_MESH = plsc.VectorSubcoreMesh(core_axis_name="c", subcore_axis_name="s")

_dispatch = functools.partial(
    pl.kernel,
    out_type=jax.ShapeDtypeStruct((SLOTS_PAD, D), jnp.float32),
    mesh=_SC_MESH,
    compiler_params=pltpu.CompilerParams(needs_layout_passes=False),
    scratch_types=[
        pltpu.VMEM((TPW,), jnp.int32),
        pltpu.VMEM((TPW,), jnp.int32),
        pltpu.VMEM((TPW, D), jnp.float32),
        pltpu.SemaphoreType.DMA,
    ],
)(_dispatch_body)

_combine = functools.partial(
    pl.kernel,
    out_type=jax.ShapeDtypeStruct((T, D), jnp.float32),
    mesh=_SC_MESH,
    compiler_params=pltpu.CompilerParams(needs_layout_passes=False),
    scratch_types=[
        pltpu.VMEM((TPW,), jnp.int32),
        pltpu.VMEM((TPW,), jnp.int32),
        pltpu.VMEM((TPW,), jnp.float32),
        pltpu.VMEM((TPW,), jnp.float32),
        pltpu.VMEM((2, TCHUNK, D), jnp.float32),
        pltpu.VMEM((2, TCHUNK, D), jnp.float32),
        pltpu.VMEM((2, TCHUNK, D), jnp.float32),
        pltpu.SemaphoreType.DMA,
        pltpu.SemaphoreType.DMA,
        pltpu.SemaphoreType.DMA,
    ],
)(_combine_body)


@jax.jit
def kernel(x, Wg, W1, b1, W2, b2, ln_g, ln_b):
    h, d1, d2, wk1, wk2 = pl.pallas_call(
        _routing_kernel,
        out_shape=[
            jax.ShapeDtypeStruct((T, D), jnp.float32),
            jax.ShapeDtypeStruct((T, 1), jnp.int32),
            jax.ShapeDtypeStruct((T, 1), jnp.int32),
            jax.ShapeDtypeStruct((T, 1), jnp.float32),
            jax.ShapeDtypeStruct((T, 1), jnp.float32),
        ],
    )(x, Wg, ln_g.reshape(1, D), ln_b.reshape(1, D))

    d1 = d1.reshape(T)
    d2 = d2.reshape(T)
    xin = _dispatch(h, d1, d2)

    b1r = b1.reshape(E, 1, FF)
    b2r = b2.reshape(E, 1, D)
    y = pl.pallas_call(
        _ffn_kernel,
        grid=(E,),
        in_specs=[
            pl.BlockSpec((CAP, D), lambda e: (e, 0)),
            pl.BlockSpec((1, D, FF), lambda e: (e, 0, 0)),
            pl.BlockSpec((1, 1, FF), lambda e: (e, 0, 0)),
            pl.BlockSpec((1, FF, D), lambda e: (e, 0, 0)),
            pl.BlockSpec((1, 1, D), lambda e: (e, 0, 0)),
        ],
        out_specs=pl.BlockSpec((CAP, D), lambda e: (e, 0)),
        out_shape=jax.ShapeDtypeStruct((SLOTS, D), jnp.float32),
        scratch_shapes=[pltpu.VMEM((CAP, FF), jnp.bfloat16)],
        compiler_params=pltpu.CompilerParams(
            vmem_limit_bytes=110 * 1024 * 1024),
    )(xin, W1, b1r, W2, b2r)

    out = _combine(y, x, d1, d2, wk1.reshape(T), wk2.reshape(T))
    return out


# speedup vs baseline: 1.0102x; 1.0102x over previous
import functools

import jax
import jax.numpy as jnp
from jax import lax
from jax.experimental import pallas as pl
from jax.experimental.pallas import tpu as pltpu
from jax.experimental.pallas import tpu_sc as plsc

E = 8
TOP_K = 2
D = 768
FF = 3072
T = 2048
CAP = int(T * TOP_K / E * 1.25)
SLOTS = E * CAP
SLOTS_PAD = SLOTS + 16

DP = D // 2
DB = 256
NJ2 = D // DB

NC, NS = 2, 16
NW = NC * NS
TPW = T // NW
TCHUNK = 16


def _routing_kernel(x_ref, wg_ref, lng_ref, lnb_ref,
                    h_ref, dsc1_ref, dsc2_ref, dg1_ref, dg2_ref,
                    w1_ref, w2_ref):
    x = x_ref[...]
    mu = jnp.mean(x, axis=-1, keepdims=True)
    var = jnp.mean((x - mu) ** 2, axis=-1, keepdims=True)
    h = (x - mu) / jnp.sqrt(var + 1e-5) * lng_ref[...] + lnb_ref[...]
    h_ref[...] = h

    logits = jnp.dot(h.astype(jnp.bfloat16), wg_ref[...].astype(jnp.bfloat16),
                     preferred_element_type=jnp.float32)
    m = jnp.max(logits, axis=-1, keepdims=True)
    ex = jnp.exp(logits - m)
    probs = ex / jnp.sum(ex, axis=-1, keepdims=True)

    iota_e = lax.broadcasted_iota(jnp.int32, (T, E), 1)
    m1 = jnp.max(probs, axis=-1, keepdims=True)
    i1 = jnp.min(jnp.where(probs == m1, iota_e, E), axis=-1, keepdims=True)
    probs2 = jnp.where(iota_e == i1, -1.0, probs)
    m2 = jnp.max(probs2, axis=-1, keepdims=True)
    i2 = jnp.min(jnp.where(probs2 == m2, iota_e, E), axis=-1, keepdims=True)

    s = jnp.clip(m1 + m2, 1e-9, None)
    g1 = m1 / s
    g2 = m2 / s

    oh1 = (iota_e == i1).astype(jnp.float32)
    oh2 = (iota_e == i2).astype(jnp.float32)
    cnt = oh1 + oh2

    r = lax.broadcasted_iota(jnp.int32, (128, 128), 0)
    c = lax.broadcasted_iota(jnp.int32, (128, 128), 1)
    ltri = (c < r).astype(jnp.float32)
    segs = []
    run = jnp.zeros((1, E), dtype=jnp.float32)
    for b in range(T // 128):
        blk = cnt[b * 128:(b + 1) * 128, :]
        excl = jnp.dot(ltri, blk, preferred_element_type=jnp.float32)
        segs.append(excl + run)
        run = run + jnp.sum(blk, axis=0, keepdims=True)
    pos = jnp.concatenate(segs, axis=0)

    p1 = jnp.sum(oh1 * pos, axis=-1, keepdims=True)
    p2 = jnp.sum(oh2 * pos, axis=-1, keepdims=True)
    keep1 = p1 < CAP
    keep2 = p2 < CAP

    slot1 = i1 * CAP + p1.astype(jnp.int32)
    slot2 = i2 * CAP + p2.astype(jnp.int32)
    dsc1_ref[...] = jnp.where(keep1, slot1, SLOTS)
    dsc2_ref[...] = jnp.where(keep2, slot2, SLOTS)
    dg1_ref[...] = jnp.where(keep1, slot1, 0)
    dg2_ref[...] = jnp.where(keep2, slot2, 0)
    w1_ref[...] = jnp.where(keep1, g1, 0.0)
    w2_ref[...] = jnp.where(keep2, g2, 0.0)


def _dispatch_body(h_hbm, d1_hbm, d2_hbm, xin_hbm, d1_v, d2_v, rows_v, sem):
    wid = lax.axis_index("s") * NC + lax.axis_index("c")
    tb = wid * TPW
    c1 = pltpu.async_copy(d1_hbm.at[pl.ds(tb, TPW)], d1_v, sem)
    c2 = pltpu.async_copy(d2_hbm.at[pl.ds(tb, TPW)], d2_v, sem)
    c3 = pltpu.async_copy(h_hbm.at[pl.ds(tb, TPW)], rows_v, sem)
    c1.wait()
    c2.wait()
    c3.wait()
    s1 = pltpu.async_copy(rows_v, xin_hbm.at[d1_v], sem)
    s2 = pltpu.async_copy(rows_v, xin_hbm.at[d2_v], sem)
    s1.wait()
    s2.wait()


def _ffn_kernel(xin_ref, w1_ref, b1_ref, w2_ref, b2_ref, y_ref, hid_ref):
    hid = jnp.dot(xin_ref[...].astype(jnp.bfloat16),
                  w1_ref[0].astype(jnp.bfloat16),
                  preferred_element_type=jnp.float32) + b1_ref[0]
    hid_ref[...] = jnp.maximum(hid, 0.0).astype(jnp.bfloat16)
    y_ref[...] = jnp.dot(hid_ref[...], w2_ref[0].astype(jnp.bfloat16),
                         preferred_element_type=jnp.float32) + b2_ref[0]


def _combine_body(y_hbm, x_hbm, dg1_hbm, dg2_hbm, w1_hbm, w2_hbm, out_hbm,
                  dg1_v, dg2_v, w1_v, w2_v, r1_v, r2_v, xb_v,
                  gsem0, gsem1, wsem):
    wid = lax.axis_index("s") * NC + lax.axis_index("c")
    tb = wid * TPW
    pltpu.sync_copy(dg1_hbm.at[pl.ds(tb, TPW)], dg1_v)
    pltpu.sync_copy(dg2_hbm.at[pl.ds(tb, TPW)], dg2_v)
    pltpu.sync_copy(w1_hbm.at[pl.ds(tb, TPW)], w1_v)
    pltpu.sync_copy(w2_hbm.at[pl.ds(tb, TPW)], w2_v)

    def fire(c):
        slot = c % 2
        sem = gsem0 if slot == 0 else gsem1
        return [
            pltpu.async_copy(y_hbm.at[dg1_v.at[pl.ds(c * TCHUNK, TCHUNK)]],
                             r1_v.at[slot], sem),
            pltpu.async_copy(y_hbm.at[dg2_v.at[pl.ds(c * TCHUNK, TCHUNK)]],
                             r2_v.at[slot], sem),
            pltpu.async_copy(x_hbm.at[pl.ds(tb + c * TCHUNK, TCHUNK)],
                             xb_v.at[slot], sem),
        ]

    nch = TPW // TCHUNK
    pend = fire(0)
    wpend = None
    for c in range(nch):
        slot = c % 2
        for d in pend:
            d.wait()
        if wpend is not None:
            wpend.wait()
            wpend = None
        if c + 1 < nch:
            pend = fire(c + 1)

        def tok_body(tloc, carry):
            sidx = jnp.zeros((16,), jnp.int32) + (c * TCHUNK + tloc)
            w1s = plsc.load_gather(w1_v, [sidx])
            w2s = plsc.load_gather(w2_v, [sidx])
            for v in range(D // 16):
                sl = pl.ds(v * 16, 16)
                acc = (xb_v[slot, tloc, sl] + w1s * r1_v[slot, tloc, sl]
                       + w2s * r2_v[slot, tloc, sl])
                xb_v[slot, tloc, sl] = acc
            return carry

        lax.fori_loop(0, TCHUNK, tok_body, 0)
        wpend = pltpu.async_copy(xb_v.at[slot],
                                 out_hbm.at[pl.ds(tb + c * TCHUNK, TCHUNK)],
                                 wsem)
    wpend.wait()


_SC_MESH = plsc.VectorSubcoreMesh(core_axis_name="c", subcore_axis_name="s")

_dispatch = functools.partial(
    pl.kernel,
    out_type=jax.ShapeDtypeStruct((SLOTS_PAD, D), jnp.float32),
    mesh=_SC_MESH,
    compiler_params=pltpu.CompilerParams(needs_layout_passes=False),
    scratch_types=[
        pltpu.VMEM((TPW,), jnp.int32),
        pltpu.VMEM((TPW,), jnp.int32),
        pltpu.VMEM((TPW, D), jnp.float32),
        pltpu.SemaphoreType.DMA,
    ],
)(_dispatch_body)

_combine = functools.partial(
    pl.kernel,
    out_type=jax.ShapeDtypeStruct((T, D), jnp.float32),
    mesh=_SC_MESH,
    compiler_params=pltpu.CompilerParams(needs_layout_passes=False),
    scratch_types=[
        pltpu.VMEM((TPW,), jnp.int32),
        pltpu.VMEM((TPW,), jnp.int32),
        pltpu.VMEM((TPW,), jnp.float32),
        pltpu.VMEM((TPW,), jnp.float32),
        pltpu.VMEM((2, TCHUNK, D), jnp.float32),
        pltpu.VMEM((2, TCHUNK, D), jnp.float32),
        pltpu.VMEM((2, TCHUNK, D), jnp.float32),
        pltpu.SemaphoreType.DMA,
        pltpu.SemaphoreType.DMA,
        pltpu.SemaphoreType.DMA,
    ],
)(_combine_body)


@jax.jit
def kernel(x, Wg, W1, b1, W2, b2, ln_g, ln_b):
    h, dsc1, dsc2, dg1, dg2, wk1, wk2 = pl.pallas_call(
        _routing_kernel,
        out_shape=[
            jax.ShapeDtypeStruct((T, D), jnp.float32),
            jax.ShapeDtypeStruct((T, 1), jnp.int32),
            jax.ShapeDtypeStruct((T, 1), jnp.int32),
            jax.ShapeDtypeStruct((T, 1), jnp.int32),
            jax.ShapeDtypeStruct((T, 1), jnp.int32),
            jax.ShapeDtypeStruct((T, 1), jnp.float32),
            jax.ShapeDtypeStruct((T, 1), jnp.float32),
        ],
    )(x, Wg, ln_g.reshape(1, D), ln_b.reshape(1, D))

    xin = _dispatch(h, dsc1.reshape(T), dsc2.reshape(T))

    b1r = b1.reshape(E, 1, FF)
    b2r = b2.reshape(E, 1, D)
    y = pl.pallas_call(
        _ffn_kernel,
        grid=(E,),
        in_specs=[
            pl.BlockSpec((CAP, D), lambda e: (e, 0)),
            pl.BlockSpec((1, D, FF), lambda e: (e, 0, 0)),
            pl.BlockSpec((1, 1, FF), lambda e: (e, 0, 0)),
            pl.BlockSpec((1, FF, D), lambda e: (e, 0, 0)),
            pl.BlockSpec((1, 1, D), lambda e: (e, 0, 0)),
        ],
        out_specs=pl.BlockSpec((CAP, D), lambda e: (e, 0)),
        out_shape=jax.ShapeDtypeStruct((SLOTS, D), jnp.float32),
        scratch_shapes=[pltpu.VMEM((CAP, FF), jnp.bfloat16)],
        compiler_params=pltpu.CompilerParams(
            vmem_limit_bytes=110 * 1024 * 1024),
    )(xin, W1, b1r, W2, b2r)

    out = _combine(y, x, dg1.reshape(T), dg2.reshape(T),
                   wk1.reshape(T), wk2.reshape(T))
    return out
